# Initial kernel scaffold; baseline (speedup 1.0000x reference)
import jax
import jax.numpy as jnp
from jax.experimental import pallas as pl


def kernel(x, embedding):
    raise NotImplementedError("write your pallas kernel here")


# speedup vs baseline: 104.6582x; 104.6582x over previous
import functools

import jax
import jax.numpy as jnp
from jax import lax
from jax.experimental import pallas as pl
from jax.experimental.pallas import tpu as pltpu
from jax.experimental.pallas import tpu_sc as plsc

_LANE = 128


def kernel(x, embedding):
    B, S = x.shape
    V, D = embedding.shape
    N = B * S
    NC, NS = 2, 16
    NW = NC * NS
    rows = N // _LANE
    rows_w = rows // NW
    R = 16
    n_g = rows_w // R

    mesh = plsc.VectorSubcoreMesh(core_axis_name="c", subcore_axis_name="s")

    @functools.partial(
        pl.kernel,
        mesh=mesh,
        out_type=(
            jax.ShapeDtypeStruct((rows, _LANE), jnp.float32),
            jax.ShapeDtypeStruct((rows, _LANE), jnp.float32),
        ),
        scratch_types=[
            pltpu.VMEM((R, _LANE), jnp.int32),
            pltpu.VMEM((R, _LANE), jnp.float32),
            pltpu.VMEM((R, _LANE), jnp.float32),
            pltpu.SemaphoreType.DMA,
        ],
    )
    def gather_k(idx_hbm, cos_hbm, sin_hbm, cos_out, sin_out,
                 idx_v, cos_v, sin_v, sem):
        wid = lax.axis_index("s") * NC + lax.axis_index("c")
        base = wid * rows_w

        def group(g, carry):
            off = base + g * R
            pltpu.sync_copy(idx_hbm.at[pl.ds(off, R)], idx_v)
            cps = []
            for j in range(R):
                cps.append(pltpu.async_copy(
                    cos_hbm.at[idx_v.at[j]], cos_v.at[j], sem))
                cps.append(pltpu.async_copy(
                    sin_hbm.at[idx_v.at[j]], sin_v.at[j], sem))
            for c in cps:
                c.wait()
            pltpu.sync_copy(cos_v, cos_out.at[pl.ds(off, R)])
            pltpu.sync_copy(sin_v, sin_out.at[pl.ds(off, R)])
            return carry

        lax.fori_loop(0, n_g, group, 0)

    cos_t = jax.lax.slice_in_dim(embedding, 0, 1, axis=1).reshape(V)
    sin_t = jax.lax.slice_in_dim(embedding, 1, 2, axis=1).reshape(V)
    cos_p, sin_p = gather_k(x.reshape(rows, _LANE), cos_t, sin_t)
    out = jnp.stack([cos_p.reshape(N), sin_p.reshape(N)], axis=-1)
    return out.reshape(B, S, D)
